# baseline (speedup 1.0000x reference)
import functools

import jax
import jax.numpy as jnp
from jax import lax
from jax.experimental import pallas as pl
from jax.experimental.pallas import tpu as pltpu
from jax.experimental.pallas import tpu_sc as plsc

VOCAB = 1000000
EMBED_DIM = 64
EMBED_PAD = 128
BATCH = 4096
SEQ = 200
B_TOTAL = BATCH * SEQ

NUM_CORES = 2
NUM_SUBCORES = 16
NUM_WORKERS = NUM_CORES * NUM_SUBCORES
B_PER_W = B_TOTAL // NUM_WORKERS

NBUF = 2
CHUNK = 128
GROUP = NBUF * CHUNK
N_GROUPS = B_PER_W // GROUP
NB = BATCH // CHUNK
assert B_PER_W % GROUP == 0

LANES = 16


def _make_emb_kernel():
    mesh = plsc.VectorSubcoreMesh(core_axis_name="c", subcore_axis_name="s")

    scratch = (
        [pltpu.VMEM((CHUNK,), jnp.int32) for _ in range(NBUF)]
        + [pltpu.VMEM((CHUNK, EMBED_PAD), jnp.float32) for _ in range(NBUF)]
        + [pltpu.VMEM((EMBED_DIM, CHUNK), jnp.float32) for _ in range(NBUF)]
        + [pltpu.SemaphoreType.DMA for _ in range(3 * NBUF)]
    )

    @functools.partial(
        pl.kernel,
        mesh=mesh,
        out_type=jax.ShapeDtypeStruct((SEQ, EMBED_DIM, BATCH), jnp.float32),
        compiler_params=pltpu.CompilerParams(needs_layout_passes=False),
        scratch_types=scratch,
    )
    def emb_kernel(idx_hbm, table_hbm, out_hbm, *scr):
        idx_vs = scr[:NBUF]
        rows_vs = scr[NBUF : 2 * NBUF]
        xp_vs = scr[2 * NBUF : 3 * NBUF]
        idx_sems = scr[3 * NBUF : 4 * NBUF]
        gat_sems = scr[4 * NBUF : 5 * NBUF]
        out_sems = scr[5 * NBUF : 6 * NBUF]

        wid = lax.axis_index("s") * NUM_CORES + lax.axis_index("c")
        base0 = wid * B_PER_W

        lane_iota = lax.iota(jnp.int32, LANES)

        def out_slot(chunk_base):
            ch = chunk_base // CHUNK
            s = ch // NB
            b0 = (ch % NB) * CHUNK
            return s, b0

        for b in range(NBUF):
            pltpu.async_copy(
                idx_hbm.at[pl.ds(base0 + b * CHUNK, CHUNK)], idx_vs[b], idx_sems[b]
            )

        def group_body(g, carry):
            base_g = base0 + g * GROUP
            for b in range(NBUF):
                pltpu.make_async_copy(
                    idx_hbm.at[pl.ds(base_g + b * CHUNK, CHUNK)],
                    idx_vs[b],
                    idx_sems[b],
                ).wait()
                pltpu.async_copy(
                    table_hbm.at[idx_vs[b]], rows_vs[b], gat_sems[b]
                )
            for b in range(NBUF):
                chunk_base = base_g + b * CHUNK
                pltpu.make_async_copy(
                    table_hbm.at[idx_vs[b]], rows_vs[b], gat_sems[b]
                ).wait()

                rows = rows_vs[b]
                xp = xp_vs[b]

                def transpose_rows(r, c, rows=rows, xp=xp):
                    rvec = jnp.full((LANES,), r, jnp.int32)
                    for k in range(EMBED_DIM // LANES):
                        val = rows[r, pl.ds(k * LANES, LANES)]
                        plsc.store_scatter(
                            xp, [lane_iota + (k * LANES), rvec], val
                        )
                    return c

                lax.fori_loop(0, CHUNK, transpose_rows, 0)

                s, b0 = out_slot(chunk_base)
                pltpu.async_copy(
                    xp, out_hbm.at[s, :, pl.ds(b0, CHUNK)], out_sems[b]
                )

                @pl.when(g + 1 < N_GROUPS)
                def _prefetch(b=b, base_g=base_g):
                    pltpu.async_copy(
                        idx_hbm.at[pl.ds(base_g + GROUP + b * CHUNK, CHUNK)],
                        idx_vs[b],
                        idx_sems[b],
                    )

            for b in range(NBUF):
                s, b0 = out_slot(base_g + b * CHUNK)
                pltpu.make_async_copy(
                    xp_vs[b],
                    out_hbm.at[s, :, pl.ds(b0, CHUNK)],
                    out_sems[b],
                ).wait()
            return carry

        lax.fori_loop(0, N_GROUPS, group_body, 0)

    return emb_kernel


_emb = _make_emb_kernel()


def kernel(x, table):
    idx = x.T.reshape(-1).astype(jnp.int32)
    table_pad = jnp.pad(table, ((0, 0), (0, EMBED_PAD - EMBED_DIM)))
    out_t = _emb(idx, table_pad)
    return jnp.transpose(out_t, (2, 0, 1))


# speedup vs baseline: 1.4059x; 1.4059x over previous
import functools

import jax
import jax.numpy as jnp
from jax import lax
from jax.experimental import pallas as pl
from jax.experimental.pallas import tpu as pltpu
from jax.experimental.pallas import tpu_sc as plsc

VOCAB = 1000000
EMBED_DIM = 64
BATCH = 4096
SEQ = 200
B_TOTAL = BATCH * SEQ

NUM_CORES = 2
NUM_SUBCORES = 16
NUM_WORKERS = NUM_CORES * NUM_SUBCORES
B_PER_W = B_TOTAL // NUM_WORKERS

NBUF = 2
CHUNK = 512
GROUP = NBUF * CHUNK
N_GROUPS = B_PER_W // GROUP
assert B_PER_W % GROUP == 0


def _make_emb_kernel():
    mesh = plsc.VectorSubcoreMesh(core_axis_name="c", subcore_axis_name="s")

    scratch = (
        [pltpu.VMEM((CHUNK,), jnp.int32) for _ in range(NBUF)]
        + [pltpu.VMEM((CHUNK, EMBED_DIM), jnp.float32) for _ in range(NBUF)]
        + [pltpu.SemaphoreType.DMA for _ in range(3 * NBUF)]
    )

    @functools.partial(
        pl.kernel,
        mesh=mesh,
        out_type=jax.ShapeDtypeStruct((B_TOTAL, EMBED_DIM), jnp.float32),
        compiler_params=pltpu.CompilerParams(use_tc_tiling_on_sc=False),
        scratch_types=scratch,
    )
    def emb_kernel(idx_hbm, table_hbm, out_hbm, *scr):
        idx_vs = scr[:NBUF]
        rows_vs = scr[NBUF : 2 * NBUF]
        idx_sems = scr[2 * NBUF : 3 * NBUF]
        gat_sems = scr[3 * NBUF : 4 * NBUF]
        out_sems = scr[4 * NBUF : 5 * NBUF]

        wid = lax.axis_index("s") * NUM_CORES + lax.axis_index("c")
        base0 = wid * B_PER_W

        for b in range(NBUF):
            pltpu.async_copy(
                idx_hbm.at[pl.ds(base0 + b * CHUNK, CHUNK)], idx_vs[b], idx_sems[b]
            )

        def group_body(g, carry):
            base_g = base0 + g * GROUP
            for b in range(NBUF):
                pltpu.make_async_copy(
                    idx_hbm.at[pl.ds(base_g + b * CHUNK, CHUNK)],
                    idx_vs[b],
                    idx_sems[b],
                ).wait()
                pltpu.async_copy(
                    table_hbm.at[idx_vs[b]], rows_vs[b], gat_sems[b]
                )
            for b in range(NBUF):
                chunk_base = base_g + b * CHUNK
                pltpu.make_async_copy(
                    table_hbm.at[idx_vs[b]], rows_vs[b], gat_sems[b]
                ).wait()
                pltpu.async_copy(
                    rows_vs[b], out_hbm.at[pl.ds(chunk_base, CHUNK)], out_sems[b]
                )

                @pl.when(g + 1 < N_GROUPS)
                def _prefetch(b=b, base_g=base_g):
                    pltpu.async_copy(
                        idx_hbm.at[pl.ds(base_g + GROUP + b * CHUNK, CHUNK)],
                        idx_vs[b],
                        idx_sems[b],
                    )

            for b in range(NBUF):
                pltpu.make_async_copy(
                    rows_vs[b],
                    out_hbm.at[pl.ds(base_g + b * CHUNK, CHUNK)],
                    out_sems[b],
                ).wait()
            return carry

        lax.fori_loop(0, N_GROUPS, group_body, 0)

    return emb_kernel


_emb = _make_emb_kernel()


def kernel(x, table):
    idx = x.reshape(-1).astype(jnp.int32)
    out = _emb(idx, table)
    return out.reshape(BATCH, SEQ, EMBED_DIM)


# speedup vs baseline: 1.4115x; 1.0040x over previous
import functools

import jax
import jax.numpy as jnp
from jax import lax
from jax.experimental import pallas as pl
from jax.experimental.pallas import tpu as pltpu
from jax.experimental.pallas import tpu_sc as plsc

VOCAB = 1000000
EMBED_DIM = 64
BATCH = 4096
SEQ = 200
B_TOTAL = BATCH * SEQ

NUM_CORES = 2
NUM_SUBCORES = 16
NUM_WORKERS = NUM_CORES * NUM_SUBCORES
B_PER_W = B_TOTAL // NUM_WORKERS

NBUF = 2
CHUNK = 512
GROUP = NBUF * CHUNK
N_GROUPS = B_PER_W // GROUP
assert B_PER_W % GROUP == 0


def _make_emb_kernel():
    mesh = plsc.VectorSubcoreMesh(core_axis_name="c", subcore_axis_name="s")

    scratch = (
        [pltpu.VMEM((CHUNK,), jnp.int32) for _ in range(NBUF)]
        + [pltpu.VMEM((CHUNK, EMBED_DIM), jnp.float32) for _ in range(NBUF)]
        + [pltpu.SemaphoreType.DMA for _ in range(3 * NBUF)]
    )

    @functools.partial(
        pl.kernel,
        mesh=mesh,
        out_type=jax.ShapeDtypeStruct((B_TOTAL, EMBED_DIM), jnp.float32),
        compiler_params=pltpu.CompilerParams(
            use_tc_tiling_on_sc=False, skip_device_barrier=True
        ),
        scratch_types=scratch,
    )
    def emb_kernel(idx_hbm, table_hbm, out_hbm, *scr):
        idx_vs = scr[:NBUF]
        rows_vs = scr[NBUF : 2 * NBUF]
        idx_sems = scr[2 * NBUF : 3 * NBUF]
        gat_sems = scr[3 * NBUF : 4 * NBUF]
        out_sems = scr[4 * NBUF : 5 * NBUF]

        wid = lax.axis_index("s") * NUM_CORES + lax.axis_index("c")
        base0 = wid * B_PER_W

        for b in range(NBUF):
            pltpu.async_copy(
                idx_hbm.at[pl.ds(base0 + b * CHUNK, CHUNK)], idx_vs[b], idx_sems[b]
            )

        def group_body(g, carry):
            base_g = base0 + g * GROUP
            for b in range(NBUF):
                pltpu.make_async_copy(
                    idx_hbm.at[pl.ds(base_g + b * CHUNK, CHUNK)],
                    idx_vs[b],
                    idx_sems[b],
                ).wait()
                pltpu.async_copy(
                    table_hbm.at[idx_vs[b]], rows_vs[b], gat_sems[b]
                )
            for b in range(NBUF):
                chunk_base = base_g + b * CHUNK
                pltpu.make_async_copy(
                    table_hbm.at[idx_vs[b]], rows_vs[b], gat_sems[b]
                ).wait()
                pltpu.async_copy(
                    rows_vs[b], out_hbm.at[pl.ds(chunk_base, CHUNK)], out_sems[b]
                )

                @pl.when(g + 1 < N_GROUPS)
                def _prefetch(b=b, base_g=base_g):
                    pltpu.async_copy(
                        idx_hbm.at[pl.ds(base_g + GROUP + b * CHUNK, CHUNK)],
                        idx_vs[b],
                        idx_sems[b],
                    )

            for b in range(NBUF):
                pltpu.make_async_copy(
                    rows_vs[b],
                    out_hbm.at[pl.ds(base_g + b * CHUNK, CHUNK)],
                    out_sems[b],
                ).wait()
            return carry

        lax.fori_loop(0, N_GROUPS, group_body, 0)

    return emb_kernel


_emb = _make_emb_kernel()


def kernel(x, table):
    idx = x.reshape(-1).astype(jnp.int32)
    out = _emb(idx, table)
    return out.reshape(BATCH, SEQ, EMBED_DIM)
